# Initial kernel scaffold; baseline (speedup 1.0000x reference)
import jax
import jax.numpy as jnp
from jax.experimental import pallas as pl


def kernel(input, weight):
    raise NotImplementedError("write your pallas kernel here")


# speedup vs baseline: 1.8380x; 1.8380x over previous
import functools

import jax
import jax.numpy as jnp
from jax import lax
from jax.experimental import pallas as pl
from jax.experimental.pallas import tpu as pltpu
from jax.experimental.pallas import tpu_sc as plsc

_D = 64
_NC = 2
_NS = 16
_NW = _NC * _NS
_CH = 512


@functools.lru_cache(maxsize=None)
def _make_gather(B):
    assert B % _NW == 0
    bpw = B // _NW
    assert bpw % _CH == 0
    nch = bpw // _CH

    mesh = plsc.VectorSubcoreMesh(core_axis_name="c", subcore_axis_name="s")

    @functools.partial(
        pl.kernel,
        out_type=jax.ShapeDtypeStruct((B, _D), jnp.float32),
        mesh=mesh,
        scratch_types=[
            pltpu.VMEM((bpw,), jnp.int32),
            pltpu.VMEM((_CH, _D), jnp.float32),
            pltpu.SemaphoreType.DMA,
        ],
        compiler_params=pltpu.CompilerParams(use_tc_tiling_on_sc=False),
    )
    def kern(idx_hbm, table_hbm, out_hbm, idx_v, rows_v, sem):
        wid = lax.axis_index("s") * _NC + lax.axis_index("c")
        base = wid * bpw
        pltpu.sync_copy(idx_hbm.at[pl.ds(base, bpw)], idx_v)

        def chunk(c, carry):
            off = c * _CH
            pltpu.async_copy(
                table_hbm.at[idx_v.at[pl.ds(off, _CH)]], rows_v, sem
            ).wait()
            pltpu.sync_copy(rows_v, out_hbm.at[pl.ds(base + off, _CH)])
            return carry

        lax.fori_loop(0, nch, chunk, 0)

    return kern


def kernel(input, weight):
    b0, b1 = input.shape
    idx = input.reshape(-1).astype(jnp.int32)
    out = _make_gather(idx.shape[0])(idx, weight)
    return out.reshape(b0, b1, _D)


# speedup vs baseline: 1.8775x; 1.0214x over previous
import functools

import jax
import jax.numpy as jnp
from jax import lax
from jax.experimental import pallas as pl
from jax.experimental.pallas import tpu as pltpu
from jax.experimental.pallas import tpu_sc as plsc

_D = 64
_NC = 2
_NS = 16
_NW = _NC * _NS
_CH = 256
_NBUF = 4
_LA = 2


@functools.lru_cache(maxsize=None)
def _make_gather(B):
    assert B % _NW == 0
    bpw = B // _NW
    assert bpw % (_CH * _NBUF) == 0
    nch = bpw // _CH
    ngrp = nch // _NBUF

    mesh = plsc.VectorSubcoreMesh(core_axis_name="c", subcore_axis_name="s")

    @functools.partial(
        pl.kernel,
        out_type=jax.ShapeDtypeStruct((B, _D), jnp.float32),
        mesh=mesh,
        scratch_types=[
            pltpu.VMEM((bpw,), jnp.int32),
            pltpu.VMEM((_NBUF, _CH, _D), jnp.float32),
            [pltpu.SemaphoreType.DMA] * _NBUF,
            [pltpu.SemaphoreType.DMA] * _NBUF,
        ],
        compiler_params=pltpu.CompilerParams(use_tc_tiling_on_sc=False),
    )
    def kern(idx_hbm, table_hbm, out_hbm, idx_v, rows_v, gsems, osems):
        wid = lax.axis_index("s") * _NC + lax.axis_index("c")
        base = wid * bpw
        pltpu.sync_copy(idx_hbm.at[pl.ds(base, bpw)], idx_v)

        def start_gather(b, c):
            pltpu.make_async_copy(
                table_hbm.at[idx_v.at[pl.ds(c * _CH, _CH)]],
                rows_v.at[b], gsems[b],
            ).start()

        def wait_gather(b):
            pltpu.make_async_copy(
                table_hbm.at[idx_v.at[pl.ds(0, _CH)]],
                rows_v.at[b], gsems[b],
            ).wait()

        def start_out(b, c):
            pltpu.make_async_copy(
                rows_v.at[b], out_hbm.at[pl.ds(base + c * _CH, _CH)], osems[b],
            ).start()

        def wait_out(b, c):
            pltpu.make_async_copy(
                rows_v.at[b], out_hbm.at[pl.ds(base + c * _CH, _CH)], osems[b],
            ).wait()

        for b in range(_LA):
            start_gather(b, b)

        def group(g, carry):
            for b in range(_NBUF):
                c = g * _NBUF + b
                bg = (b + _LA) % _NBUF

                @pl.when(c + _LA < nch)
                def _():
                    @pl.when(c + _LA >= _NBUF)
                    def _():
                        wait_out(bg, c + _LA - _NBUF)
                    start_gather(bg, c + _LA)

                wait_gather(b)
                start_out(b, c)
            return carry

        lax.fori_loop(0, ngrp, group, 0)

        for b in range(_NBUF):
            wait_out(b, nch - _NBUF + b)

    return kern


def kernel(input, weight):
    b0, b1 = input.shape
    idx = input.reshape(-1).astype(jnp.int32)
    out = _make_gather(idx.shape[0])(idx, weight)
    return out.reshape(b0, b1, _D)


# speedup vs baseline: 2.4132x; 1.2853x over previous
import functools

import jax
import jax.numpy as jnp
from jax import lax
from jax.experimental import pallas as pl
from jax.experimental.pallas import tpu as pltpu
from jax.experimental.pallas import tpu_sc as plsc

_D = 64
_DP = 128
_NC = 2
_NS = 16
_NW = _NC * _NS
_CH = 128
_NBUF = 4
_LA = 2


@functools.lru_cache(maxsize=None)
def _make_gather(B):
    assert B % _NW == 0
    bpw = B // _NW
    assert bpw % (_CH * _NBUF) == 0
    nch = bpw // _CH
    ngrp = nch // _NBUF

    mesh = plsc.VectorSubcoreMesh(core_axis_name="c", subcore_axis_name="s")

    @functools.partial(
        pl.kernel,
        out_type=jax.ShapeDtypeStruct((B, _DP), jnp.float32),
        mesh=mesh,
        scratch_types=[
            pltpu.VMEM((bpw,), jnp.int32),
            pltpu.VMEM((_NBUF, _CH, _DP), jnp.float32),
            [pltpu.SemaphoreType.DMA] * _NBUF,
            [pltpu.SemaphoreType.DMA] * _NBUF,
        ],
    )
    def kern(idx_hbm, table_hbm, out_hbm, idx_v, rows_v, gsems, osems):
        wid = lax.axis_index("s") * _NC + lax.axis_index("c")
        base = wid * bpw
        pltpu.sync_copy(idx_hbm.at[pl.ds(base, bpw)], idx_v)

        def start_gather(b, c):
            pltpu.make_async_copy(
                table_hbm.at[idx_v.at[pl.ds(c * _CH, _CH)]],
                rows_v.at[b], gsems[b],
            ).start()

        def wait_gather(b):
            pltpu.make_async_copy(
                table_hbm.at[idx_v.at[pl.ds(0, _CH)]],
                rows_v.at[b], gsems[b],
            ).wait()

        def start_out(b, c):
            pltpu.make_async_copy(
                rows_v.at[b],
                out_hbm.at[pl.ds(base + c * _CH, _CH)], osems[b],
            ).start()

        def wait_out(b, c):
            pltpu.make_async_copy(
                rows_v.at[b],
                out_hbm.at[pl.ds(base + c * _CH, _CH)], osems[b],
            ).wait()

        for b in range(_LA):
            start_gather(b, b)

        def group(g, carry):
            for b in range(_NBUF):
                c = g * _NBUF + b
                bg = (b + _LA) % _NBUF

                @pl.when(c + _LA < nch)
                def _():
                    @pl.when(c + _LA >= _NBUF)
                    def _():
                        wait_out(bg, c + _LA - _NBUF)
                    start_gather(bg, c + _LA)

                wait_gather(b)
                start_out(b, c)
            return carry

        lax.fori_loop(0, ngrp, group, 0)

        for b in range(_NBUF):
            wait_out(b, nch - _NBUF + b)

    return kern


def kernel(input, weight):
    b0, b1 = input.shape
    table = jnp.pad(weight, ((0, 0), (0, _DP - _D)))
    idx = input.T.reshape(-1).astype(jnp.int32)
    out = _make_gather(idx.shape[0])(idx, table)
    return out.reshape(b1, b0, _DP)[:, :, :_D].transpose(1, 0, 2)
